# Initial kernel scaffold; baseline (speedup 1.0000x reference)
import jax
import jax.numpy as jnp
from jax.experimental import pallas as pl


def kernel(features, table):
    raise NotImplementedError("write your pallas kernel here")


# speedup vs baseline: 6.2814x; 6.2814x over previous
import functools

import jax
import jax.numpy as jnp
from jax import lax
from jax.experimental import pallas as pl
from jax.experimental.pallas import tpu as pltpu
from jax.experimental.pallas import tpu_sc as plsc

NC = 2
NS = 16
NW = NC * NS

B = 16384 * 26
D = 32
BPW = B // NW
C = 1664
NCHUNK = BPW // C


def _body(feat_hbm, table_hbm, out_hbm, idx_v, buf_v, g0, g1, w0, w1):
    wid = lax.axis_index("s") * NC + lax.axis_index("c")
    base = wid * BPW

    pltpu.sync_copy(feat_hbm.at[pl.ds(base, BPW)], idx_v)

    gsem = (g0, g1)
    wsem = (w0, w1)

    def start_gather(c, b):
        return pltpu.async_copy(
            table_hbm.at[idx_v.at[pl.ds(c * C, C)]], buf_v.at[b], gsem[b]
        )

    def start_write(c, b):
        return pltpu.async_copy(
            buf_v.at[b], out_hbm.at[pl.ds(base + c * C, C)], wsem[b]
        )

    gh = {0: start_gather(0, 0)}
    wh = {}
    for c in range(NCHUNK):
        b = c & 1
        if c + 1 < NCHUNK:
            nb = (c + 1) & 1
            if c - 1 >= 0:
                wh.pop(c - 1).wait()
            gh[c + 1] = start_gather(c + 1, nb)
        gh.pop(c).wait()
        wh[c] = start_write(c, b)
    for c in sorted(wh):
        wh.pop(c).wait()


@jax.jit
def _lookup(feats, table):
    mesh = plsc.VectorSubcoreMesh(core_axis_name="c", subcore_axis_name="s")
    return pl.kernel(
        _body,
        out_type=jax.ShapeDtypeStruct((B, D), jnp.float32),
        mesh=mesh,
        compiler_params=pltpu.CompilerParams(use_tc_tiling_on_sc=False),
        scratch_types=[
            pltpu.VMEM((BPW,), jnp.int32),
            pltpu.VMEM((2, C, D), jnp.float32),
            pltpu.SemaphoreType.DMA,
            pltpu.SemaphoreType.DMA,
            pltpu.SemaphoreType.DMA,
            pltpu.SemaphoreType.DMA,
        ],
    )(feats, table)


def kernel(features, table):
    feats = features.reshape(-1).astype(jnp.int32)
    t = table.at[0].set(0.0)
    out = _lookup(feats, t)
    return out.reshape(features.shape + (D,))
